# baseline (speedup 1.0000x reference)
import functools

import jax
import jax.numpy as jnp
from jax import lax
from jax.experimental import pallas as pl
from jax.experimental.pallas import tpu as pltpu
from jax.experimental.pallas import tpu_sc as plsc

_INFO = plsc.get_sparse_core_info()
_NC = _INFO.num_cores
_NS = _INFO.num_subcores
_NW = _NC * _NS

_B = 4 * 2048
_D = 1728
_BPW = _B // _NW
_C = 32
_NCH = _BPW // _C
_NBUF = 2


def _make_gather(vocab, d):
    mesh = plsc.VectorSubcoreMesh(core_axis_name="c", subcore_axis_name="s")

    @functools.partial(
        pl.kernel,
        out_type=jax.ShapeDtypeStruct((_B, d), jnp.float32),
        mesh=mesh,
        compiler_params=pltpu.CompilerParams(use_tc_tiling_on_sc=False),
        scratch_types=[
            pltpu.VMEM((_NCH, _C), jnp.int32),
            pltpu.VMEM((_NBUF, _C, d), jnp.float32),
            pltpu.SemaphoreType.DMA,
            pltpu.SemaphoreType.DMA,
            pltpu.SemaphoreType.DMA,
            pltpu.SemaphoreType.DMA,
        ],
    )
    def gather_kernel(idx_hbm, table_hbm, out_hbm, idx_v, bufs, g0, g1, p0, p1):
        wid = lax.axis_index("s") * _NC + lax.axis_index("c")
        base = wid * _BPW

        pltpu.sync_copy(idx_hbm.at[wid], idx_v)

        gsem = (g0, g1)
        psem = (p0, p1)
        h_g = [None] * _NBUF
        h_p = [None] * _NBUF

        h_g[0] = pltpu.async_copy(
            table_hbm.at[idx_v.at[0]], bufs.at[0], gsem[0]
        )
        for g in range(_NCH):
            b = g % _NBUF
            nb = (g + 1) % _NBUF
            if g + 1 < _NCH:
                if h_p[nb] is not None:
                    h_p[nb].wait()
                h_g[nb] = pltpu.async_copy(
                    table_hbm.at[idx_v.at[g + 1]], bufs.at[nb], gsem[nb]
                )
            h_g[b].wait()
            h_p[b] = pltpu.async_copy(
                bufs.at[b], out_hbm.at[pl.ds(base + g * _C, _C)], psem[b]
            )
        for b in range(_NBUF):
            if h_p[b] is not None:
                h_p[b].wait()

    return gather_kernel


@jax.jit
def kernel(input_ids, weight):
    vocab, d = weight.shape
    ids = input_ids.reshape(_NW, _NCH, _C).astype(jnp.int32)
    out = _make_gather(vocab, d)(ids, weight)
    return out.reshape(input_ids.shape + (d,))


# speedup vs baseline: 3.8887x; 3.8887x over previous
import functools

import jax
import jax.numpy as jnp
from jax import lax
from jax.experimental import pallas as pl
from jax.experimental.pallas import tpu as pltpu
from jax.experimental.pallas import tpu_sc as plsc

_INFO = plsc.get_sparse_core_info()
_NC = _INFO.num_cores
_NS = _INFO.num_subcores
_NW = _NC * _NS

_B = 4 * 2048
_D = 1728
_DM = 1664
_BPW = _B // _NW
_C = 32
_NCH = _BPW // _C
_NBUF = 2


def _make_gather(vocab, d):
    mesh = plsc.VectorSubcoreMesh(core_axis_name="c", subcore_axis_name="s")

    @functools.partial(
        pl.kernel,
        out_type=jax.ShapeDtypeStruct((_B, d), jnp.float32),
        mesh=mesh,
        compiler_params=pltpu.CompilerParams(use_tc_tiling_on_sc=True),
        scratch_types=[
            pltpu.VMEM((_NCH, _C), jnp.int32),
            pltpu.VMEM((_NBUF, _C, _D), jnp.float32),
            pltpu.VMEM((_NBUF, _C, 128), jnp.float32),
            pltpu.SemaphoreType.DMA,
            pltpu.SemaphoreType.DMA,
            pltpu.SemaphoreType.DMA,
            pltpu.SemaphoreType.DMA,
        ],
    )
    def gather_kernel(idx_hbm, table_hbm, tail_hbm, out_hbm, idx_v, bufs,
                      tails, g0, g1, p0, p1):
        wid = lax.axis_index("s") * _NC + lax.axis_index("c")
        base = wid * _BPW

        pltpu.sync_copy(idx_hbm.at[wid], idx_v)

        gsem = (g0, g1)
        psem = (p0, p1)

        def start_gather(g, b):
            hm = pltpu.async_copy(
                table_hbm.at[idx_v.at[g], pl.ds(0, _DM)],
                bufs.at[b].at[:, pl.ds(0, _DM)], gsem[b]
            )
            ht = pltpu.async_copy(
                tail_hbm.at[idx_v.at[g]], tails.at[b], gsem[b],
            )
            return hm, ht

        def fixup(b):
            def row(r, carry):
                for j in range(4):
                    v = tails[b, r, pl.ds(16 * j, 16)]
                    bufs[b, r, pl.ds(_DM + 16 * j, 16)] = v
                return carry
            lax.fori_loop(0, _C, row, 0)

        def start_put(g, b):
            rows = pl.ds(base + g * _C, _C)
            hm = pltpu.async_copy(bufs.at[b], out_hbm.at[rows], psem[b])
            return (hm,)

        h_g = [None] * _NBUF
        h_p = [None] * _NBUF

        h_g[0] = start_gather(0, 0)
        for g in range(_NCH):
            b = g % _NBUF
            nb = (g + 1) % _NBUF
            if g + 1 < _NCH:
                if h_p[nb] is not None:
                    for h in h_p[nb]:
                        h.wait()
                h_g[nb] = start_gather(g + 1, nb)
            for h in h_g[b]:
                h.wait()
            fixup(b)
            h_p[b] = start_put(g, b)
        for b in range(_NBUF):
            if h_p[b] is not None:
                for h in h_p[b]:
                    h.wait()

    return gather_kernel


@jax.jit
def kernel(input_ids, weight):
    vocab, d = weight.shape
    ids = input_ids.reshape(_NW, _NCH, _C).astype(jnp.int32)
    tail = jnp.pad(weight[:, _DM:], ((0, 0), (0, 128 - (d - _DM))))
    out = _make_gather(vocab, d)(ids, weight, tail)
    return out.reshape(input_ids.shape + (d,))


# speedup vs baseline: 4.4235x; 1.1375x over previous
import functools

import jax
import jax.numpy as jnp
from jax import lax
from jax.experimental import pallas as pl
from jax.experimental.pallas import tpu as pltpu
from jax.experimental.pallas import tpu_sc as plsc

_INFO = plsc.get_sparse_core_info()
_NC = _INFO.num_cores
_NS = _INFO.num_subcores
_NW = _NC * _NS

_BATCH = 4
_T = 2048
_B = _BATCH * _T
_F = 1728
_V = 23236
_NJT = _F // 8
_JPW = (_NJT + _NW - 1) // _NW

_CW = 1664
_CW13 = 1536
_VT = 13 * _CW + _CW13
_NCHK = 15


def _make_kernel():
    mesh = plsc.VectorSubcoreMesh(core_axis_name="c", subcore_axis_name="s")

    @functools.partial(
        pl.kernel,
        out_type=jax.ShapeDtypeStruct((_BATCH, _F, _T), jnp.float32),
        mesh=mesh,
        compiler_params=pltpu.CompilerParams(use_tc_tiling_on_sc=True, needs_layout_passes=False),
        scratch_types=[
            pltpu.VMEM((_B,), jnp.int32),
            pltpu.VMEM((_B,), jnp.int32),
            pltpu.VMEM((16,), jnp.int32),
            pltpu.VMEM((2, 8, _CW + 64), jnp.float32),
            pltpu.VMEM((8, 192), jnp.float32),
            pltpu.VMEM((8, _B), jnp.float32),
            pltpu.VMEM((16,), jnp.int32),
            pltpu.SemaphoreType.DMA,
            pltpu.SemaphoreType.DMA,
            pltpu.SemaphoreType.DMA,
        ],
    )
    def lane_kernel(wt_hbm, wtail_hbm, sidx_hbm, stok_hbm, bnd_hbm, out_hbm,
                    sidxv, stokv, idx8, tb, tbl2, oall, bnd, g0, g1, psem):
        wid = lax.axis_index("s") * _NC + lax.axis_index("c")

        pltpu.sync_copy(sidx_hbm, sidxv)
        pltpu.sync_copy(stok_hbm, stokv)
        pltpu.sync_copy(bnd_hbm, bnd)

        gsem = (g0, g1)
        rows = [jnp.full((16,), s, jnp.int32) for s in range(8)]

        lane = lax.iota(jnp.int32, 16)

        def chunk_copy(k, par):
            r8 = idx8.at[pl.ds(0, 8)]
            base = jnp.minimum(k * _CW, _VT - _CW)
            return pltpu.make_async_copy(
                wt_hbm.at[r8, pl.ds(base, _CW)],
                tb.at[par].at[:, pl.ds(0, _CW)], gsem[par],
            )

        def process(k, buf, buf_base):
            bv = bnd[pl.ds(0, 16)]
            lo_pos = jnp.sum(jnp.where(lane == k, bv, 0))
            hi_pos = jnp.sum(jnp.where(lane == k + 1, bv, 0))
            lo = lo_pos // 16
            hi = (hi_pos + 15) // 16

            def group(g, carry):
                p = g * 16
                sv = sidxv[pl.ds(p, 16)]
                tv = stokv[pl.ds(p, 16)]
                pos = p + lane
                m = (pos >= lo_pos) & (pos < hi_pos)
                col = sv - buf_base
                for s in range(8):
                    vals = plsc.load_gather(buf, [rows[s], col], mask=m)
                    plsc.store_scatter(oall, [rows[s], tv], vals, mask=m)
                return carry

            lax.fori_loop(lo, hi, group, 0)

        for i in range(_JPW):
            jt = wid + _NW * i

            @pl.when(jt < _NJT)
            def _():
                idx8[...] = jt * 8 + lax.iota(jnp.int32, 16)
                chunk_copy(0, 0).start()

                def pair(k2, carry):
                    k = 2 * k2
                    chunk_copy(k + 1, 1).start()
                    chunk_copy(k, 0).wait()
                    process(k, tb.at[0], jnp.minimum(k * _CW, _VT - _CW))

                    @pl.when(k + 2 <= 13)
                    def _():
                        chunk_copy(k + 2, 0).start()

                    chunk_copy(k + 1, 1).wait()
                    process(k + 1, tb.at[1],
                            jnp.minimum((k + 1) * _CW, _VT - _CW))
                    return carry

                lax.fori_loop(0, 7, pair, 0)

                ht = pltpu.async_copy(
                    wtail_hbm.at[idx8.at[pl.ds(0, 8)]],
                    tbl2.at[:, pl.ds(0, 128)], gsem[0],
                )
                ht.wait()
                process(14, tbl2, _VT)

                hp = []
                for b in range(_BATCH):
                    hp.append(pltpu.async_copy(
                        oall.at[:, pl.ds(b * _T, _T)],
                        out_hbm.at[b, pl.ds(jt * 8, 8)], psem,
                    ))
                for hh in hp:
                    hh.wait()

    return lane_kernel


@jax.jit
def kernel(input_ids, weight):
    wt = weight.T
    wtail = jnp.pad(wt[:, _VT:], ((0, 0), (0, 128 - (_V - _VT))))
    idx = input_ids.reshape(-1).astype(jnp.int32)
    sidx, stok = lax.sort_key_val(idx, lax.iota(jnp.int32, _B))
    edges = jnp.minimum(jnp.arange(_NCHK + 1, dtype=jnp.int32) * _CW, _V)
    edges = edges.at[14].set(_VT)
    bnd = jnp.searchsorted(sidx, edges).astype(jnp.int32)
    out3 = _make_kernel()(wt, wtail, sidx, stok, bnd)
    return jnp.transpose(out3, (0, 2, 1))


# speedup vs baseline: 4.8107x; 1.0875x over previous
import functools

import jax
import jax.numpy as jnp
from jax import lax
from jax.experimental import pallas as pl
from jax.experimental.pallas import tpu as pltpu
from jax.experimental.pallas import tpu_sc as plsc

_INFO = plsc.get_sparse_core_info()
_NC = _INFO.num_cores
_NS = _INFO.num_subcores
_NW = _NC * _NS

_BATCH = 4
_T = 2048
_B = _BATCH * _T
_F = 1728
_V = 23236
_NJT = _F // 8
_JPW = (_NJT + _NW - 1) // _NW

_CW = 2560
_CWB = 2624
_VT = 9 * _CW
_TAILW = 256


def _make_kernel():
    mesh = plsc.VectorSubcoreMesh(core_axis_name="c", subcore_axis_name="s")

    @functools.partial(
        pl.kernel,
        out_type=jax.ShapeDtypeStruct((_BATCH, _F, _T), jnp.float32),
        mesh=mesh,
        compiler_params=pltpu.CompilerParams(
            use_tc_tiling_on_sc=True, needs_layout_passes=False),
        scratch_types=[
            pltpu.VMEM((_B,), jnp.int32),
            pltpu.VMEM((_B,), jnp.int32),
            pltpu.VMEM((16,), jnp.int32),
            pltpu.VMEM((2, 8, _CWB), jnp.float32),
            pltpu.VMEM((8, _TAILW + 64), jnp.float32),
            pltpu.VMEM((8, _B), jnp.float32),
            pltpu.VMEM((16,), jnp.int32),
            pltpu.SemaphoreType.DMA,
            pltpu.SemaphoreType.DMA,
            pltpu.SemaphoreType.DMA,
            pltpu.SemaphoreType.DMA,
        ],
    )
    def lane_kernel(wt_hbm, wtail_hbm, sidx_hbm, stok_hbm, bnd_hbm, out_hbm,
                    sidxv, stokv, idx8, tb, tbl2, oall, bnd, g0, g1, g2, psem):
        wid = lax.axis_index("s") * _NC + lax.axis_index("c")

        pltpu.sync_copy(sidx_hbm, sidxv)
        pltpu.sync_copy(stok_hbm, stokv)
        pltpu.sync_copy(bnd_hbm, bnd)

        gsem = (g0, g1)
        rows = [jnp.full((16,), s, jnp.int32) for s in range(8)]
        lane = lax.iota(jnp.int32, 16)

        def chunk_copy(k, par):
            r8 = idx8.at[pl.ds(0, 8)]
            return pltpu.make_async_copy(
                wt_hbm.at[r8, pl.ds(k * _CW, _CW)],
                tb.at[par].at[:, pl.ds(0, _CW)], gsem[par],
            )

        def process(k, buf, base):
            bv = bnd[pl.ds(0, 16)]
            lo_pos = jnp.sum(jnp.where(lane == k, bv, 0))
            hi_pos = jnp.sum(jnp.where(lane == k + 1, bv, 0))

            def qstep(q, carry):
                for hh in range(2):
                    p = q * 32 + hh * 16
                    sv = sidxv[pl.ds(p, 16)]
                    tv = stokv[pl.ds(p, 16)]
                    pos = p + lane
                    m = (pos >= lo_pos) & (pos < hi_pos)
                    col = sv - base
                    for s in range(8):
                        vals = plsc.load_gather(buf, [rows[s], col], mask=m)
                        plsc.store_scatter(oall, [rows[s], tv], vals, mask=m)
                return carry

            lax.fori_loop(lo_pos // 32, (hi_pos + 31) // 32, qstep, 0)

        def drain_out():
            for b in range(_BATCH):
                pltpu.make_async_copy(
                    oall.at[:, pl.ds(b * _T, _T)],
                    out_hbm.at[b, pl.ds(0, 8)], psem,
                ).wait()

        for i in range(_JPW):
            jt = wid + _NW * i

            @pl.when(jt < _NJT)
            def _():
                idx8[...] = jt * 8 + lax.iota(jnp.int32, 16)
                ht = pltpu.async_copy(
                    wtail_hbm.at[idx8.at[pl.ds(0, 8)]],
                    tbl2.at[:, pl.ds(0, _TAILW)], g2,
                )
                chunk_copy(0, 0).start()
                chunk_copy(1, 1).start()
                if i > 0:
                    drain_out()

                def pair(k2, carry):
                    k = 2 * k2
                    chunk_copy(k, 0).wait()
                    process(k, tb.at[0], k * _CW)

                    @pl.when(k + 2 <= 8)
                    def _():
                        chunk_copy(k + 2, 0).start()

                    chunk_copy(k + 1, 1).wait()
                    process(k + 1, tb.at[1], (k + 1) * _CW)

                    @pl.when(k + 3 <= 8)
                    def _():
                        chunk_copy(k + 3, 1).start()

                    return carry

                lax.fori_loop(0, 4, pair, 0)

                chunk_copy(8, 0).wait()
                process(8, tb.at[0], 8 * _CW)
                ht.wait()
                process(9, tbl2, _VT)

                for b in range(_BATCH):
                    pltpu.async_copy(
                        oall.at[:, pl.ds(b * _T, _T)],
                        out_hbm.at[b, pl.ds(jt * 8, 8)], psem,
                    )

        drain_out()

    return lane_kernel


@jax.jit
def kernel(input_ids, weight):
    wt = weight.T
    wtail = jnp.pad(wt[:, _VT:], ((0, 0), (0, _TAILW - (_V - _VT))))
    idx = input_ids.reshape(-1).astype(jnp.int32)
    sidx, stok = lax.sort_key_val(idx, lax.iota(jnp.int32, _B))
    edges = jnp.minimum(jnp.arange(16, dtype=jnp.int32) * _CW, _V)
    bnd = jnp.searchsorted(sidx, edges).astype(jnp.int32)
    out3 = _make_kernel()(wt, wtail, sidx, stok, bnd)
    return jnp.transpose(out3, (0, 2, 1))
